# baseline (speedup 1.0000x reference)
import jax
import jax.numpy as jnp
from jax.experimental import pallas as pl
from jax.experimental.pallas import tpu as pltpu


def _dma_body(t_ref, o_ref, sem):
    cp = pltpu.make_async_copy(t_ref, o_ref, sem)
    cp.start()
    cp.wait()


def kernel(x, table):
    seq_len = x.shape[1]
    d_model = table.shape[1]
    out = pl.pallas_call(
        _dma_body,
        in_specs=[pl.BlockSpec(memory_space=pl.ANY)],
        out_specs=pl.BlockSpec(memory_space=pl.ANY),
        out_shape=jax.ShapeDtypeStruct((seq_len, d_model), table.dtype),
        scratch_shapes=[pltpu.SemaphoreType.DMA],
    )(table[:seq_len])
    return out[None]


# speedup vs baseline: 24.1575x; 24.1575x over previous
import functools

import jax
import jax.numpy as jnp
from jax import lax
from jax.experimental import pallas as pl
from jax.experimental.pallas import tpu as pltpu
from jax.experimental.pallas import tpu_sc as plsc

_INFO = plsc.get_sparse_core_info()
_NC = _INFO.num_cores
_NS = _INFO.num_subcores
_NW = _NC * _NS

_CHUNK = 8
_NBUF = 7
_AHEAD = 3


def _copy_body(rows_per_w, table_hbm, out_hbm, buf, gsem, ssem):
    wid = lax.axis_index("s") * _NC + lax.axis_index("c")
    base = wid * rows_per_w
    nchunks = rows_per_w // _CHUNK

    def gather(j):
        g = pltpu.make_async_copy(
            table_hbm.at[pl.ds(base + j * _CHUNK, _CHUNK)], buf.at[j % _NBUF],
            gsem)
        g.start()
        return g

    gaths, scats = [], []
    for j in range(min(_AHEAD, nchunks)):
        gaths.append(gather(j))
    for i in range(nchunks):
        gaths[i].wait()
        s = pltpu.make_async_copy(
            buf.at[i % _NBUF], out_hbm.at[pl.ds(base + i * _CHUNK, _CHUNK)],
            ssem)
        s.start()
        scats.append(s)
        j = i + _AHEAD
        if j < nchunks:
            if j >= _NBUF:
                scats[j - _NBUF].wait()
            gaths.append(gather(j))
    for s in scats[-min(_NBUF, nchunks):]:
        s.wait()


def kernel(x, table):
    seq_len = x.shape[1]
    d_model = table.shape[1]
    rows_per_w = seq_len // _NW
    mesh = plsc.VectorSubcoreMesh(core_axis_name="c", subcore_axis_name="s")
    out = pl.kernel(
        functools.partial(_copy_body, rows_per_w),
        out_type=jax.ShapeDtypeStruct((seq_len, d_model), table.dtype),
        mesh=mesh,
        scratch_types=[
            pltpu.VMEM((_NBUF, _CHUNK, d_model), table.dtype),
            pltpu.SemaphoreType.DMA,
            pltpu.SemaphoreType.DMA,
        ],
    )(table[:seq_len])
    return out[None]
